# baseline (speedup 1.0000x reference)
import functools

import jax
import jax.numpy as jnp
from jax import lax
from jax.experimental import pallas as pl
from jax.experimental.pallas import tpu as pltpu
from jax.experimental.pallas import tpu_sc as plsc

C = 256
NH = 8
DH = C // NH
NL = 4
NP_ = 4
HGRID = 64
NQ = HGRID * HGRID
NV = NL * NQ
WF = 4

QBLK = 512
NQB = NQ // QBLK

NPAIR = NQ * NH
NCONTRIB = NL * NP_ * 4

NTILE = 32
NPT = NPAIR // NTILE
GB = 16
NBLK = NPT // GB


def _ln(x, g, b):
    m = jnp.mean(x, axis=-1, keepdims=True)
    v = jnp.mean((x - m) ** 2, axis=-1, keepdims=True)
    return (x - m) / jnp.sqrt(v + 1e-5) * g + b


def _qkv_body(x_ref, pos_ref, g_ref, b_ref, w_ref, o_ref):
    xn = _ln(x_ref[...] + pos_ref[...], g_ref[...], b_ref[...])
    res = jnp.dot(xn, w_ref[...], preferred_element_type=jnp.float32)
    for k in range(3 * NH):
        o_ref[k] = res[:, k * DH:(k + 1) * DH]


def _tc_qkv(x, pos, g, b, w):
    return pl.pallas_call(
        _qkv_body,
        grid=(NQB,),
        in_specs=[
            pl.BlockSpec((QBLK, C), lambda i: (i, 0)),
            pl.BlockSpec((QBLK, C), lambda i: (i, 0)),
            pl.BlockSpec((1, C), lambda i: (0, 0)),
            pl.BlockSpec((1, C), lambda i: (0, 0)),
            pl.BlockSpec((C, 3 * C), lambda i: (0, 0)),
        ],
        out_specs=pl.BlockSpec((3 * NH, QBLK, DH), lambda i: (0, i, 0)),
        out_shape=jax.ShapeDtypeStruct((3 * NH, NQ, DH), jnp.float32),
    )(x, pos, g, b, w)


def _attn_body(q_ref, k_ref, v_ref, o_ref):
    q = q_ref[0]
    k = k_ref[0]
    s = lax.dot_general(q, k, (((1,), (1,)), ((), ())),
                        preferred_element_type=jnp.float32) * (DH ** -0.5)
    m = jnp.max(s, axis=-1, keepdims=True)
    e = jnp.exp(s - m)
    z = jnp.sum(e, axis=-1, keepdims=True)
    a = e / z
    o_ref[0] = jnp.dot(a, v_ref[0], preferred_element_type=jnp.float32)


def _tc_attn(qh, kh, vh):
    return pl.pallas_call(
        _attn_body,
        grid=(NH, NQB),
        in_specs=[
            pl.BlockSpec((1, QBLK, DH), lambda h, i: (h, i, 0)),
            pl.BlockSpec((1, NQ, DH), lambda h, i: (h, 0, 0)),
            pl.BlockSpec((1, NQ, DH), lambda h, i: (h, 0, 0)),
        ],
        out_specs=pl.BlockSpec((1, QBLK, DH), lambda h, i: (h, i, 0)),
        out_shape=jax.ShapeDtypeStruct((NH, NQ, DH), jnp.float32),
    )(qh, kh, vh)


def _proj_res_body(a_ref, w_ref, b_ref, r_ref, o_ref):
    a = jnp.concatenate([a_ref[h] for h in range(NH)], axis=-1)
    o_ref[...] = (jnp.dot(a, w_ref[...], preferred_element_type=jnp.float32)
                  + b_ref[...] + r_ref[...])


def _tc_proj_res(attnh, w, b, res):
    return pl.pallas_call(
        _proj_res_body,
        grid=(NQB,),
        in_specs=[
            pl.BlockSpec((NH, QBLK, DH), lambda i: (0, i, 0)),
            pl.BlockSpec((C, C), lambda i: (0, 0)),
            pl.BlockSpec((1, C), lambda i: (0, 0)),
            pl.BlockSpec((QBLK, C), lambda i: (i, 0)),
        ],
        out_specs=pl.BlockSpec((QBLK, C), lambda i: (i, 0)),
        out_shape=jax.ShapeDtypeStruct((NQ, C), jnp.float32),
    )(attnh, w, b, res)


def _value_body(h_ref, g_ref, b_ref, w_ref, bv_ref, o_ref):
    xn = _ln(h_ref[...], g_ref[...], b_ref[...])
    res = jnp.dot(xn, w_ref[...], preferred_element_type=jnp.float32) + bv_ref[...]
    for h in range(NH):
        o_ref[h, 0] = res[:, h * DH:(h + 1) * DH]


def _tc_value(hist, g, b, w, bv):
    blk = 1024
    return pl.pallas_call(
        _value_body,
        grid=(NV // blk,),
        in_specs=[
            pl.BlockSpec((blk, C), lambda i: (i, 0)),
            pl.BlockSpec((1, C), lambda i: (0, 0)),
            pl.BlockSpec((1, C), lambda i: (0, 0)),
            pl.BlockSpec((C, C), lambda i: (0, 0)),
            pl.BlockSpec((1, C), lambda i: (0, 0)),
        ],
        out_specs=pl.BlockSpec((NH, 1, blk, DH), lambda i: (0, i // 4, i % 4, 0)),
        out_shape=jax.ShapeDtypeStruct((NH, NL, NQ, DH), jnp.float32),
    )(hist, g, b, w, bv)


def _corner_body(v_ref, o_ref):
    v = v_ref[0, 0].reshape(HGRID, HGRID, DH)
    sx = jnp.concatenate([v[:, 1:, :], v[:, HGRID - 1:, :]], axis=1)
    sy = jnp.concatenate([v[1:, :, :], v[HGRID - 1:, :, :]], axis=0)
    sxy = jnp.concatenate([sx[1:, :, :], sx[HGRID - 1:, :, :]], axis=0)
    o_ref[0, 0] = jnp.concatenate([v, sx, sy, sxy], axis=-1).reshape(NQ, 4 * DH)


def _tc_corner_pack(vplanes):
    return pl.pallas_call(
        _corner_body,
        grid=(NH, NL),
        in_specs=[pl.BlockSpec((1, 1, NQ, DH), lambda h, l: (h, l, 0, 0))],
        out_specs=pl.BlockSpec((1, 1, NQ, 4 * DH), lambda h, l: (h, l, 0, 0)),
        out_shape=jax.ShapeDtypeStruct((NH, NL, NQ, 4 * DH), jnp.float32),
    )(vplanes)


NLANE = NH * NL * NP_


def _samp_body(x_ref, pos_ref, g_ref, b_ref, wso_ref, bso_ref, wa_ref, ba_ref,
               ref_ref, idx_ref, w_ref):
    xq = _ln(x_ref[...] + pos_ref[...], g_ref[...], b_ref[...])
    so = (jnp.dot(xq, wso_ref[...], preferred_element_type=jnp.float32)
          + bso_ref[...])
    sx = so[:, :NLANE]
    sy = so[:, NLANE:]
    logits = (jnp.dot(xq, wa_ref[...], preferred_element_type=jnp.float32)
              + ba_ref[...])
    parts = []
    for h in range(NH):
        blk = logits[:, h * 16:(h + 1) * 16]
        m = jnp.max(blk, axis=-1, keepdims=True)
        e = jnp.exp(blk - m)
        parts.append(e / jnp.sum(e, axis=-1, keepdims=True))
    aw = jnp.concatenate(parts, axis=-1)

    lane_l = (lax.broadcasted_iota(jnp.int32, (NL, NLANE), 1) // NP_) % NL
    m4 = (lane_l == lax.broadcasted_iota(jnp.int32, (NL, NLANE), 0)
          ).astype(jnp.float32)
    rx = jnp.dot(ref_ref[..., 0], m4, preferred_element_type=jnp.float32)
    ry = jnp.dot(ref_ref[..., 1], m4, preferred_element_type=jnp.float32)

    gx = (rx + sx * (1.0 / HGRID)) * HGRID - 0.5
    gy = (ry + sy * (1.0 / HGRID)) * HGRID - 0.5
    x0 = jnp.floor(gx)
    y0 = jnp.floor(gy)
    wx1 = gx - x0
    wx0 = 1.0 - wx1
    wy1 = gy - y0
    wy0 = 1.0 - wy1
    bx = jnp.clip(x0, 0.0, HGRID - 2.0)
    by = jnp.clip(y0, 0.0, HGRID - 2.0)

    lane = lax.broadcasted_iota(jnp.int32, (QBLK, NLANE), 1)
    hl = lane // 16 * NL + (lane // NP_) % NL
    idx_ref[...] = (hl * NQ + by.astype(jnp.int32) * HGRID
                    + bx.astype(jnp.int32))

    rr = lax.broadcasted_iota(jnp.int32, (NLANE, 4 * NLANE), 0)
    cc = lax.broadcasted_iota(jnp.int32, (NLANE, 4 * NLANE), 1)
    acc = jnp.zeros((QBLK, 4 * NLANE), jnp.float32)
    for s, (dy, dx) in enumerate(((0.0, 0.0), (0.0, 1.0), (1.0, 0.0), (1.0, 1.0))):
        sxc = bx + dx
        syc = by + dy
        fx = jnp.where(sxc == x0, wx0, jnp.where(sxc == x0 + 1.0, wx1, 0.0))
        fy = jnp.where(syc == y0, wy0, jnp.where(syc == y0 + 1.0, wy1, 0.0))
        ws = fx * fy * aw
        perm = (cc == (rr // 16) * 64 + s * 16 + rr % 16).astype(jnp.float32)
        acc = acc + jnp.dot(ws, perm, preferred_element_type=jnp.float32)
    w_ref[...] = acc


def _tc_samp(x, pos, g, b, wso, bso, wa, ba, ref):
    return pl.pallas_call(
        _samp_body,
        grid=(NQB,),
        in_specs=[
            pl.BlockSpec((QBLK, C), lambda i: (i, 0)),
            pl.BlockSpec((QBLK, C), lambda i: (i, 0)),
            pl.BlockSpec((1, C), lambda i: (0, 0)),
            pl.BlockSpec((1, C), lambda i: (0, 0)),
            pl.BlockSpec((C, 2 * NLANE), lambda i: (0, 0)),
            pl.BlockSpec((1, 2 * NLANE), lambda i: (0, 0)),
            pl.BlockSpec((C, NLANE), lambda i: (0, 0)),
            pl.BlockSpec((1, NLANE), lambda i: (0, 0)),
            pl.BlockSpec((QBLK, NL, 2), lambda i: (i, 0, 0)),
        ],
        out_specs=[
            pl.BlockSpec((QBLK, NLANE), lambda i: (i, 0)),
            pl.BlockSpec((QBLK, 4 * NLANE), lambda i: (i, 0)),
        ],
        out_shape=[
            jax.ShapeDtypeStruct((NQ, NLANE), jnp.int32),
            jax.ShapeDtypeStruct((NQ, 4 * NLANE), jnp.float32),
        ],
    )(x, pos, g, b, wso, bso, wa, ba, ref)


def _outffn_body(s_ref, wout_ref, bout_ref, x_ref, g_ref, b_ref,
                 w1_ref, b1_ref, w2_ref, b2_ref, o_ref):
    x2 = (jnp.dot(s_ref[...], wout_ref[...], preferred_element_type=jnp.float32)
          + bout_ref[...] + x_ref[...])
    xn = _ln(x2, g_ref[...], b_ref[...])
    h1 = jnp.dot(xn, w1_ref[...], preferred_element_type=jnp.float32) + b1_ref[...]
    ff = jnp.dot(h1, w2_ref[...], preferred_element_type=jnp.float32) + b2_ref[...]
    o_ref[...] = ff + x2


def _tc_outffn(sampled, wout, bout, x, g, b, w1, b1, w2, b2):
    return pl.pallas_call(
        _outffn_body,
        grid=(NQB,),
        in_specs=[
            pl.BlockSpec((QBLK, C), lambda i: (i, 0)),
            pl.BlockSpec((C, C), lambda i: (0, 0)),
            pl.BlockSpec((1, C), lambda i: (0, 0)),
            pl.BlockSpec((QBLK, C), lambda i: (i, 0)),
            pl.BlockSpec((1, C), lambda i: (0, 0)),
            pl.BlockSpec((1, C), lambda i: (0, 0)),
            pl.BlockSpec((C, WF * C), lambda i: (0, 0)),
            pl.BlockSpec((1, WF * C), lambda i: (0, 0)),
            pl.BlockSpec((WF * C, C), lambda i: (0, 0)),
            pl.BlockSpec((1, C), lambda i: (0, 0)),
        ],
        out_specs=pl.BlockSpec((QBLK, C), lambda i: (i, 0)),
        out_shape=jax.ShapeDtypeStruct((NQ, C), jnp.float32),
    )(sampled, wout, bout, x, g, b, w1, b1, w2, b2)


def _sc_body(value_hbm, idx_hbm, w_hbm, out_hbm, idx_v, w_v, rows_v, out_v,
             sem_i, sem_w, sem_r):
    wid = lax.axis_index("s") * 2 + lax.axis_index("c")
    brow = wid * NBLK

    def fire_idx(b, slot):
        pltpu.async_copy(idx_hbm.at[brow + b], idx_v.at[slot], sem_i)
        pltpu.async_copy(w_hbm.at[brow + b], w_v.at[slot], sem_w)

    def wait_idx(slot):
        pltpu.make_async_copy(idx_hbm.at[0], idx_v.at[slot], sem_i).wait()
        pltpu.make_async_copy(w_hbm.at[0], w_v.at[slot], sem_w).wait()

    def fire_gathers(slot):
        def fj(j, c):
            pltpu.async_copy(value_hbm.at[idx_v.at[slot, pl.ds(j * 16, 16)]],
                             rows_v.at[slot, pl.ds(j * 16, 16)], sem_r)
            return c
        lax.fori_loop(0, GB, fj, 0)

    def drain_gathers(slot):
        def dj(j, c):
            pltpu.make_async_copy(
                value_hbm.at[idx_v.at[slot, pl.ds(j * 16, 16)]],
                rows_v.at[slot, pl.ds(j * 16, 16)], sem_r).wait()
            return c
        lax.fori_loop(0, GB, dj, 0)

    def compute_block(b, slot):
        dnums = lax.GatherDimensionNumbers(
            offset_dims=(), collapsed_slice_dims=(0,), start_index_map=(0,))

        def pj(j, c):
            wvecs = [w_v[slot, pl.ds(j * 64 + g * 16, 16)] for g in range(4)]
            acc0 = jnp.zeros((16,), jnp.float32)
            acc1 = jnp.zeros((16,), jnp.float32)
            for i in range(NCONTRIB):
                g, lane = divmod(i, 16)
                ws = lax.gather(
                    wvecs[g], jnp.full((16, 1), lane, jnp.int32), dnums, (1,),
                    mode=lax.GatherScatterMode.PROMISE_IN_BOUNDS)
                r0 = rows_v[slot, j * 16 + i % 16, pl.ds((i // 16) * DH, 16)]
                r1 = rows_v[slot, j * 16 + i % 16, pl.ds((i // 16) * DH + 16, 16)]
                acc0 = acc0 + ws * r0
                acc1 = acc1 + ws * r1
            pit = b * GB + j
            out_v[pit // 4, pl.ds((pit % 4) * DH, 16)] = acc0
            out_v[pit // 4, pl.ds((pit % 4) * DH + 16, 16)] = acc1
            return c
        lax.fori_loop(0, GB, pj, 0)

    def body_seq(b, c):
        fire_idx(b, 0)
        wait_idx(0)
        fire_gathers(0)
        drain_gathers(0)
        compute_block(b, 0)
        return c

    lax.fori_loop(0, NBLK, body_seq, 0)
    pltpu.sync_copy(out_v, out_hbm.at[pl.ds(wid * (NPT // 4), NPT // 4)])


def _sc_gather(valtab, idx, w):
    mesh = plsc.VectorSubcoreMesh(core_axis_name="c", subcore_axis_name="s")
    fn = functools.partial(
        pl.kernel,
        out_type=jax.ShapeDtypeStruct((NPAIR // 4, 4 * DH), jnp.float32),
        mesh=mesh,
        scratch_types=[
            pltpu.VMEM((2, GB * 16), jnp.int32),
            pltpu.VMEM((2, GB * NCONTRIB), jnp.float32),
            pltpu.VMEM((2, GB * 16, 4 * DH), jnp.float32),
            pltpu.VMEM((NPT // 4, 4 * DH), jnp.float32),
            pltpu.SemaphoreType.DMA,
            pltpu.SemaphoreType.DMA,
            pltpu.SemaphoreType.DMA,
        ],
    )(_sc_body)
    return fn(valtab, idx, w)


def kernel(query, history_bevs, reference_points, spatial_shapes,
           level_start_index, pos_embedding, params):
    p = params
    q2 = query[0]
    pos2 = pos_embedding[0]
    hist2 = history_bevs[0]

    def r2(v):
        return v.reshape(1, -1)

    qkvh = _tc_qkv(q2, pos2, r2(p['ln1_g']), r2(p['ln1_b']), p['Wqkv'])
    attnh = _tc_attn(qkvh[:NH], qkvh[NH:2 * NH], qkvh[2 * NH:])
    x = _tc_proj_res(attnh, p['Wo'], r2(p['bo']), q2)

    vplanes = _tc_value(hist2, r2(p['ln2_g']), r2(p['ln2_b']), p['Wv'],
                        r2(p['bv']))
    table4 = _tc_corner_pack(vplanes).reshape(NH * NL * NQ, 4 * DH)

    wso_p = jnp.concatenate([p['Wso'][:, 0::2], p['Wso'][:, 1::2]], axis=1)
    bso_p = jnp.concatenate([p['bso'][0::2], p['bso'][1::2]])
    idxq, wq = _tc_samp(x, pos2, r2(p['ln2_g']), r2(p['ln2_b']),
                        wso_p, r2(bso_p), p['Wa'], r2(p['ba']),
                        reference_points[0])

    sampled = _sc_gather(table4, idxq.reshape(NPAIR // GB, GB * 16),
                         wq.reshape(NPAIR // GB, GB * NCONTRIB)).reshape(NQ, C)

    out = _tc_outffn(sampled, p['Wout'], r2(p['bout']), x,
                     r2(p['ln3_g']), r2(p['ln3_b']),
                     p['W1'], r2(p['b1']), p['W2'], r2(p['b2']))
    return out[None]


# speedup vs baseline: 1.1223x; 1.1223x over previous
import functools

import jax
import jax.numpy as jnp
from jax import lax
from jax.experimental import pallas as pl
from jax.experimental.pallas import tpu as pltpu
from jax.experimental.pallas import tpu_sc as plsc

C = 256
NH = 8
DH = C // NH
NL = 4
NP_ = 4
HGRID = 64
NQ = HGRID * HGRID
NV = NL * NQ
WF = 4

QBLK = 512
NQB = NQ // QBLK

NPAIR = NQ * NH
NCONTRIB = NL * NP_ * 4

NTILE = 32
NPT = NPAIR // NTILE
GB = 16
NBLK = NPT // GB


def _ln(x, g, b):
    m = jnp.mean(x, axis=-1, keepdims=True)
    v = jnp.mean((x - m) ** 2, axis=-1, keepdims=True)
    return (x - m) / jnp.sqrt(v + 1e-5) * g + b


def _qkv_body(x_ref, pos_ref, g_ref, b_ref, w_ref, o_ref):
    xn = _ln(x_ref[...] + pos_ref[...], g_ref[...], b_ref[...])
    res = jnp.dot(xn, w_ref[...], preferred_element_type=jnp.float32)
    for k in range(3 * NH):
        o_ref[k] = res[:, k * DH:(k + 1) * DH]


def _tc_qkv(x, pos, g, b, w):
    return pl.pallas_call(
        _qkv_body,
        grid=(NQB,),
        in_specs=[
            pl.BlockSpec((QBLK, C), lambda i: (i, 0)),
            pl.BlockSpec((QBLK, C), lambda i: (i, 0)),
            pl.BlockSpec((1, C), lambda i: (0, 0)),
            pl.BlockSpec((1, C), lambda i: (0, 0)),
            pl.BlockSpec((C, 3 * C), lambda i: (0, 0)),
        ],
        out_specs=pl.BlockSpec((3 * NH, QBLK, DH), lambda i: (0, i, 0)),
        out_shape=jax.ShapeDtypeStruct((3 * NH, NQ, DH), jnp.float32),
    )(x, pos, g, b, w)


def _attn_body(q_ref, k_ref, v_ref, o_ref):
    q = q_ref[0]
    k = k_ref[0]
    s = lax.dot_general(q, k, (((1,), (1,)), ((), ())),
                        preferred_element_type=jnp.float32) * (DH ** -0.5)
    m = jnp.max(s, axis=-1, keepdims=True)
    e = jnp.exp(s - m)
    z = jnp.sum(e, axis=-1, keepdims=True)
    a = e / z
    o_ref[0] = jnp.dot(a, v_ref[0], preferred_element_type=jnp.float32)


def _tc_attn(qh, kh, vh):
    return pl.pallas_call(
        _attn_body,
        grid=(NH, NQB),
        in_specs=[
            pl.BlockSpec((1, QBLK, DH), lambda h, i: (h, i, 0)),
            pl.BlockSpec((1, NQ, DH), lambda h, i: (h, 0, 0)),
            pl.BlockSpec((1, NQ, DH), lambda h, i: (h, 0, 0)),
        ],
        out_specs=pl.BlockSpec((1, QBLK, DH), lambda h, i: (h, i, 0)),
        out_shape=jax.ShapeDtypeStruct((NH, NQ, DH), jnp.float32),
    )(qh, kh, vh)


def _proj_res_body(a_ref, w_ref, b_ref, r_ref, o_ref):
    a = jnp.concatenate([a_ref[h] for h in range(NH)], axis=-1)
    o_ref[...] = (jnp.dot(a, w_ref[...], preferred_element_type=jnp.float32)
                  + b_ref[...] + r_ref[...])


def _tc_proj_res(attnh, w, b, res):
    return pl.pallas_call(
        _proj_res_body,
        grid=(NQB,),
        in_specs=[
            pl.BlockSpec((NH, QBLK, DH), lambda i: (0, i, 0)),
            pl.BlockSpec((C, C), lambda i: (0, 0)),
            pl.BlockSpec((1, C), lambda i: (0, 0)),
            pl.BlockSpec((QBLK, C), lambda i: (i, 0)),
        ],
        out_specs=pl.BlockSpec((QBLK, C), lambda i: (i, 0)),
        out_shape=jax.ShapeDtypeStruct((NQ, C), jnp.float32),
    )(attnh, w, b, res)


def _value_body(h_ref, g_ref, b_ref, w_ref, bv_ref, o_ref):
    xn = _ln(h_ref[...], g_ref[...], b_ref[...])
    res = jnp.dot(xn, w_ref[...], preferred_element_type=jnp.float32) + bv_ref[...]
    for h in range(NH):
        o_ref[h, 0] = res[:, h * DH:(h + 1) * DH]


def _tc_value(hist, g, b, w, bv):
    blk = 1024
    return pl.pallas_call(
        _value_body,
        grid=(NV // blk,),
        in_specs=[
            pl.BlockSpec((blk, C), lambda i: (i, 0)),
            pl.BlockSpec((1, C), lambda i: (0, 0)),
            pl.BlockSpec((1, C), lambda i: (0, 0)),
            pl.BlockSpec((C, C), lambda i: (0, 0)),
            pl.BlockSpec((1, C), lambda i: (0, 0)),
        ],
        out_specs=pl.BlockSpec((NH, 1, blk, DH), lambda i: (0, i // 4, i % 4, 0)),
        out_shape=jax.ShapeDtypeStruct((NH, NL, NQ, DH), jnp.float32),
    )(hist, g, b, w, bv)


def _corner_body(v_ref, o_ref):
    v = v_ref[0, 0].reshape(HGRID, HGRID, DH)
    sx = jnp.concatenate([v[:, 1:, :], v[:, HGRID - 1:, :]], axis=1)
    sy = jnp.concatenate([v[1:, :, :], v[HGRID - 1:, :, :]], axis=0)
    sxy = jnp.concatenate([sx[1:, :, :], sx[HGRID - 1:, :, :]], axis=0)
    o_ref[0, 0] = jnp.concatenate([v, sx, sy, sxy], axis=-1).reshape(NQ, 4 * DH)


def _tc_corner_pack(vplanes):
    return pl.pallas_call(
        _corner_body,
        grid=(NH, NL),
        in_specs=[pl.BlockSpec((1, 1, NQ, DH), lambda h, l: (h, l, 0, 0))],
        out_specs=pl.BlockSpec((1, 1, NQ, 4 * DH), lambda h, l: (h, l, 0, 0)),
        out_shape=jax.ShapeDtypeStruct((NH, NL, NQ, 4 * DH), jnp.float32),
    )(vplanes)


NLANE = NH * NL * NP_


def _samp_body(x_ref, pos_ref, g_ref, b_ref, wso_ref, bso_ref, wa_ref, ba_ref,
               ref_ref, idx_ref, w_ref):
    xq = _ln(x_ref[...] + pos_ref[...], g_ref[...], b_ref[...])
    so = (jnp.dot(xq, wso_ref[...], preferred_element_type=jnp.float32,
                  precision=lax.Precision.HIGHEST)
          + bso_ref[...])
    sx = so[:, :NLANE]
    sy = so[:, NLANE:]
    logits = (jnp.dot(xq, wa_ref[...], preferred_element_type=jnp.float32)
              + ba_ref[...])
    parts = []
    for h in range(NH):
        blk = logits[:, h * 16:(h + 1) * 16]
        m = jnp.max(blk, axis=-1, keepdims=True)
        e = jnp.exp(blk - m)
        parts.append(e / jnp.sum(e, axis=-1, keepdims=True))
    aw = jnp.concatenate(parts, axis=-1)

    lane_l = (lax.broadcasted_iota(jnp.int32, (NL, NLANE), 1) // NP_) % NL
    m4 = (lane_l == lax.broadcasted_iota(jnp.int32, (NL, NLANE), 0)
          ).astype(jnp.float32)
    rx = jnp.dot(ref_ref[..., 0], m4, preferred_element_type=jnp.float32)
    ry = jnp.dot(ref_ref[..., 1], m4, preferred_element_type=jnp.float32)

    gx = (rx + sx * (1.0 / HGRID)) * HGRID - 0.5
    gy = (ry + sy * (1.0 / HGRID)) * HGRID - 0.5
    x0 = jnp.floor(gx)
    y0 = jnp.floor(gy)
    wx1 = gx - x0
    wx0 = 1.0 - wx1
    wy1 = gy - y0
    wy0 = 1.0 - wy1
    bx = jnp.clip(x0, 0.0, HGRID - 2.0)
    by = jnp.clip(y0, 0.0, HGRID - 2.0)

    lane = lax.broadcasted_iota(jnp.int32, (QBLK, NLANE), 1)
    hl = lane // 16 * NL + (lane // NP_) % NL
    idx_ref[...] = (hl * NQ + by.astype(jnp.int32) * HGRID
                    + bx.astype(jnp.int32))

    rr = lax.broadcasted_iota(jnp.int32, (NLANE, 4 * NLANE), 0)
    cc = lax.broadcasted_iota(jnp.int32, (NLANE, 4 * NLANE), 1)
    acc = jnp.zeros((QBLK, 4 * NLANE), jnp.float32)
    for s, (dy, dx) in enumerate(((0.0, 0.0), (0.0, 1.0), (1.0, 0.0), (1.0, 1.0))):
        sxc = bx + dx
        syc = by + dy
        fx = jnp.where(sxc == x0, wx0, jnp.where(sxc == x0 + 1.0, wx1, 0.0))
        fy = jnp.where(syc == y0, wy0, jnp.where(syc == y0 + 1.0, wy1, 0.0))
        ws = fx * fy * aw
        perm = (cc == (rr // 16) * 64 + s * 16 + rr % 16).astype(jnp.float32)
        acc = acc + jnp.dot(ws, perm, preferred_element_type=jnp.float32)
    w_ref[...] = acc


def _tc_samp(x, pos, g, b, wso, bso, wa, ba, ref):
    return pl.pallas_call(
        _samp_body,
        grid=(NQB,),
        in_specs=[
            pl.BlockSpec((QBLK, C), lambda i: (i, 0)),
            pl.BlockSpec((QBLK, C), lambda i: (i, 0)),
            pl.BlockSpec((1, C), lambda i: (0, 0)),
            pl.BlockSpec((1, C), lambda i: (0, 0)),
            pl.BlockSpec((C, 2 * NLANE), lambda i: (0, 0)),
            pl.BlockSpec((1, 2 * NLANE), lambda i: (0, 0)),
            pl.BlockSpec((C, NLANE), lambda i: (0, 0)),
            pl.BlockSpec((1, NLANE), lambda i: (0, 0)),
            pl.BlockSpec((QBLK, NL, 2), lambda i: (i, 0, 0)),
        ],
        out_specs=[
            pl.BlockSpec((QBLK, NLANE), lambda i: (i, 0)),
            pl.BlockSpec((QBLK, 4 * NLANE), lambda i: (i, 0)),
        ],
        out_shape=[
            jax.ShapeDtypeStruct((NQ, NLANE), jnp.int32),
            jax.ShapeDtypeStruct((NQ, 4 * NLANE), jnp.float32),
        ],
    )(x, pos, g, b, wso, bso, wa, ba, ref)


def _outffn_body(s_ref, wout_ref, bout_ref, x_ref, g_ref, b_ref,
                 w1_ref, b1_ref, w2_ref, b2_ref, o_ref):
    x2 = (jnp.dot(s_ref[...], wout_ref[...], preferred_element_type=jnp.float32)
          + bout_ref[...] + x_ref[...])
    xn = _ln(x2, g_ref[...], b_ref[...])
    h1 = jnp.dot(xn, w1_ref[...], preferred_element_type=jnp.float32) + b1_ref[...]
    ff = jnp.dot(h1, w2_ref[...], preferred_element_type=jnp.float32) + b2_ref[...]
    o_ref[...] = ff + x2


def _tc_outffn(sampled, wout, bout, x, g, b, w1, b1, w2, b2):
    return pl.pallas_call(
        _outffn_body,
        grid=(NQB,),
        in_specs=[
            pl.BlockSpec((QBLK, C), lambda i: (i, 0)),
            pl.BlockSpec((C, C), lambda i: (0, 0)),
            pl.BlockSpec((1, C), lambda i: (0, 0)),
            pl.BlockSpec((QBLK, C), lambda i: (i, 0)),
            pl.BlockSpec((1, C), lambda i: (0, 0)),
            pl.BlockSpec((1, C), lambda i: (0, 0)),
            pl.BlockSpec((C, WF * C), lambda i: (0, 0)),
            pl.BlockSpec((1, WF * C), lambda i: (0, 0)),
            pl.BlockSpec((WF * C, C), lambda i: (0, 0)),
            pl.BlockSpec((1, C), lambda i: (0, 0)),
        ],
        out_specs=pl.BlockSpec((QBLK, C), lambda i: (i, 0)),
        out_shape=jax.ShapeDtypeStruct((NQ, C), jnp.float32),
    )(sampled, wout, bout, x, g, b, w1, b1, w2, b2)


def _sc_body(value_hbm, idx_hbm, w_hbm, out_hbm, idx_v, w_v, rows_v, out_v,
             sem_i, sem_w, sem_r):
    wid = lax.axis_index("s") * 2 + lax.axis_index("c")
    brow = wid * NBLK

    def fire_idx(b, slot):
        pltpu.async_copy(idx_hbm.at[brow + b], idx_v.at[slot], sem_i.at[slot])
        pltpu.async_copy(w_hbm.at[brow + b], w_v.at[slot], sem_w.at[slot])

    def wait_idx(slot):
        pltpu.make_async_copy(idx_hbm.at[0], idx_v.at[slot],
                              sem_i.at[slot]).wait()
        pltpu.make_async_copy(w_hbm.at[0], w_v.at[slot],
                              sem_w.at[slot]).wait()

    def fire_gathers(slot):
        def fj(j, c):
            pltpu.async_copy(value_hbm.at[idx_v.at[slot, pl.ds(j * 16, 16)]],
                             rows_v.at[slot, pl.ds(j * 16, 16)], sem_r.at[slot])
            return c
        lax.fori_loop(0, GB, fj, 0)

    def drain_gathers(slot):
        def dj(j, c):
            pltpu.make_async_copy(
                value_hbm.at[idx_v.at[slot, pl.ds(j * 16, 16)]],
                rows_v.at[slot, pl.ds(j * 16, 16)], sem_r.at[slot]).wait()
            return c
        lax.fori_loop(0, GB, dj, 0)

    def compute_block(b, slot):
        dnums = lax.GatherDimensionNumbers(
            offset_dims=(), collapsed_slice_dims=(0,), start_index_map=(0,))

        def pj(j, c):
            wvecs = [w_v[slot, pl.ds(j * 64 + g * 16, 16)] for g in range(4)]
            acc0 = jnp.zeros((16,), jnp.float32)
            acc1 = jnp.zeros((16,), jnp.float32)
            for i in range(NCONTRIB):
                g, lane = divmod(i, 16)
                ws = lax.gather(
                    wvecs[g], jnp.full((16, 1), lane, jnp.int32), dnums, (1,),
                    mode=lax.GatherScatterMode.PROMISE_IN_BOUNDS)
                r0 = rows_v[slot, j * 16 + i % 16, pl.ds((i // 16) * DH, 16)]
                r1 = rows_v[slot, j * 16 + i % 16, pl.ds((i // 16) * DH + 16, 16)]
                acc0 = acc0 + ws * r0
                acc1 = acc1 + ws * r1
            pit = b * GB + j
            out_v[pit // 4, pl.ds((pit % 4) * DH, 16)] = acc0
            out_v[pit // 4, pl.ds((pit % 4) * DH + 16, 16)] = acc1
            return c
        lax.fori_loop(0, GB, pj, 0)

    fire_idx(0, 0)
    wait_idx(0)
    fire_gathers(0)
    fire_idx(1, 1)

    def body2(t, c):
        b0 = 2 * t
        b1 = 2 * t + 1
        drain_gathers(0)
        wait_idx(1)
        fire_gathers(1)
        compute_block(b0, 0)

        @pl.when(b0 + 2 < NBLK)
        def _():
            fire_idx(b0 + 2, 0)

        drain_gathers(1)

        @pl.when(b1 + 1 < NBLK)
        def _():
            wait_idx(0)
            fire_gathers(0)

        compute_block(b1, 1)

        @pl.when(b1 + 2 < NBLK)
        def _():
            fire_idx(b1 + 2, 1)

        return c

    lax.fori_loop(0, NBLK // 2, body2, 0)
    pltpu.sync_copy(out_v, out_hbm.at[pl.ds(wid * (NPT // 4), NPT // 4)])


def _sc_gather(valtab, idx, w):
    mesh = plsc.VectorSubcoreMesh(core_axis_name="c", subcore_axis_name="s")
    fn = functools.partial(
        pl.kernel,
        out_type=jax.ShapeDtypeStruct((NPAIR // 4, 4 * DH), jnp.float32),
        mesh=mesh,
        scratch_types=[
            pltpu.VMEM((2, GB * 16), jnp.int32),
            pltpu.VMEM((2, GB * NCONTRIB), jnp.float32),
            pltpu.VMEM((2, GB * 16, 4 * DH), jnp.float32),
            pltpu.VMEM((NPT // 4, 4 * DH), jnp.float32),
            pltpu.SemaphoreType.DMA((2,)),
            pltpu.SemaphoreType.DMA((2,)),
            pltpu.SemaphoreType.DMA((2,)),
        ],
    )(_sc_body)
    return fn(valtab, idx, w)


def kernel(query, history_bevs, reference_points, spatial_shapes,
           level_start_index, pos_embedding, params):
    p = params
    q2 = query[0]
    pos2 = pos_embedding[0]
    hist2 = history_bevs[0]

    def r2(v):
        return v.reshape(1, -1)

    qkvh = _tc_qkv(q2, pos2, r2(p['ln1_g']), r2(p['ln1_b']), p['Wqkv'])
    attnh = _tc_attn(qkvh[:NH], qkvh[NH:2 * NH], qkvh[2 * NH:])
    x = _tc_proj_res(attnh, p['Wo'], r2(p['bo']), q2)

    vplanes = _tc_value(hist2, r2(p['ln2_g']), r2(p['ln2_b']), p['Wv'],
                        r2(p['bv']))
    table4 = _tc_corner_pack(vplanes).reshape(NH * NL * NQ, 4 * DH)

    wso_p = jnp.concatenate([p['Wso'][:, 0::2], p['Wso'][:, 1::2]], axis=1)
    bso_p = jnp.concatenate([p['bso'][0::2], p['bso'][1::2]])
    idxq, wq = _tc_samp(x, pos2, r2(p['ln2_g']), r2(p['ln2_b']),
                        wso_p, r2(bso_p), p['Wa'], r2(p['ba']),
                        reference_points[0])

    sampled = _sc_gather(table4, idxq.reshape(NPAIR // GB, GB * 16),
                         wq.reshape(NPAIR // GB, GB * NCONTRIB)).reshape(NQ, C)

    out = _tc_outffn(sampled, p['Wout'], r2(p['bout']), x,
                     r2(p['ln3_g']), r2(p['ln3_b']),
                     p['W1'], r2(p['b1']), p['W2'], r2(p['b2']))
    return out[None]


# speedup vs baseline: 1.1368x; 1.0129x over previous
import functools

import jax
import jax.numpy as jnp
from jax import lax
from jax.experimental import pallas as pl
from jax.experimental.pallas import tpu as pltpu
from jax.experimental.pallas import tpu_sc as plsc

C = 256
NH = 8
DH = C // NH
NL = 4
NP_ = 4
HGRID = 64
NQ = HGRID * HGRID
NV = NL * NQ
WF = 4

QBLK = 512
NQB = NQ // QBLK

NPAIR = NQ * NH
NCONTRIB = NL * NP_ * 4

NTILE = 32
NPT = NPAIR // NTILE
GB = 16
NBLK = NPT // GB


def _ln(x, g, b):
    m = jnp.mean(x, axis=-1, keepdims=True)
    v = jnp.mean((x - m) ** 2, axis=-1, keepdims=True)
    return (x - m) / jnp.sqrt(v + 1e-5) * g + b


def _qkv_body(x_ref, pos_ref, g_ref, b_ref, w_ref, o_ref):
    xn = _ln(x_ref[...] + pos_ref[...], g_ref[...], b_ref[...])
    res = jnp.dot(xn, w_ref[...], preferred_element_type=jnp.float32)
    for k in range(3 * NH):
        o_ref[k] = res[:, k * DH:(k + 1) * DH].astype(jnp.bfloat16)


def _tc_qkv(x, pos, g, b, w):
    return pl.pallas_call(
        _qkv_body,
        grid=(NQB,),
        in_specs=[
            pl.BlockSpec((QBLK, C), lambda i: (i, 0)),
            pl.BlockSpec((QBLK, C), lambda i: (i, 0)),
            pl.BlockSpec((1, C), lambda i: (0, 0)),
            pl.BlockSpec((1, C), lambda i: (0, 0)),
            pl.BlockSpec((C, 3 * C), lambda i: (0, 0)),
        ],
        out_specs=pl.BlockSpec((3 * NH, QBLK, DH), lambda i: (0, i, 0)),
        out_shape=jax.ShapeDtypeStruct((3 * NH, NQ, DH), jnp.bfloat16),
    )(x, pos, g, b, w)


def _attn_body(q_ref, k_ref, v_ref, o_ref):
    q = q_ref[0]
    k = k_ref[0]
    s = lax.dot_general(q, k, (((1,), (1,)), ((), ())),
                        preferred_element_type=jnp.float32) * (DH ** -0.5)
    m = jnp.max(s, axis=-1, keepdims=True)
    e = jnp.exp(s - m)
    z = jnp.sum(e, axis=-1, keepdims=True)
    a = (e / z).astype(jnp.bfloat16)
    o_ref[0] = jnp.dot(a, v_ref[0], preferred_element_type=jnp.float32)


def _tc_attn(qh, kh, vh):
    return pl.pallas_call(
        _attn_body,
        grid=(NH, NQB),
        in_specs=[
            pl.BlockSpec((1, QBLK, DH), lambda h, i: (h, i, 0)),
            pl.BlockSpec((1, NQ, DH), lambda h, i: (h, 0, 0)),
            pl.BlockSpec((1, NQ, DH), lambda h, i: (h, 0, 0)),
        ],
        out_specs=pl.BlockSpec((1, QBLK, DH), lambda h, i: (h, i, 0)),
        out_shape=jax.ShapeDtypeStruct((NH, NQ, DH), jnp.float32),
    )(qh, kh, vh)


def _proj_res_body(a_ref, w_ref, b_ref, r_ref, o_ref):
    a = jnp.concatenate([a_ref[h] for h in range(NH)], axis=-1)
    o_ref[...] = (jnp.dot(a, w_ref[...], preferred_element_type=jnp.float32)
                  + b_ref[...] + r_ref[...])


def _tc_proj_res(attnh, w, b, res):
    return pl.pallas_call(
        _proj_res_body,
        grid=(NQB,),
        in_specs=[
            pl.BlockSpec((NH, QBLK, DH), lambda i: (0, i, 0)),
            pl.BlockSpec((C, C), lambda i: (0, 0)),
            pl.BlockSpec((1, C), lambda i: (0, 0)),
            pl.BlockSpec((QBLK, C), lambda i: (i, 0)),
        ],
        out_specs=pl.BlockSpec((QBLK, C), lambda i: (i, 0)),
        out_shape=jax.ShapeDtypeStruct((NQ, C), jnp.float32),
    )(attnh, w, b, res)


def _value_body(h_ref, g_ref, b_ref, w_ref, bv_ref, o_ref):
    xn = _ln(h_ref[...], g_ref[...], b_ref[...])
    res = jnp.dot(xn, w_ref[...], preferred_element_type=jnp.float32) + bv_ref[...]
    for h in range(NH):
        o_ref[h, 0] = res[:, h * DH:(h + 1) * DH]


def _tc_value(hist, g, b, w, bv):
    blk = 1024
    return pl.pallas_call(
        _value_body,
        grid=(NV // blk,),
        in_specs=[
            pl.BlockSpec((blk, C), lambda i: (i, 0)),
            pl.BlockSpec((1, C), lambda i: (0, 0)),
            pl.BlockSpec((1, C), lambda i: (0, 0)),
            pl.BlockSpec((C, C), lambda i: (0, 0)),
            pl.BlockSpec((1, C), lambda i: (0, 0)),
        ],
        out_specs=pl.BlockSpec((NH, 1, blk, DH), lambda i: (0, i // 4, i % 4, 0)),
        out_shape=jax.ShapeDtypeStruct((NH, NL, NQ, DH), jnp.float32),
    )(hist, g, b, w, bv)


def _corner_body(v_ref, o_ref):
    v = v_ref[0, 0].reshape(HGRID, HGRID, DH)
    sx = jnp.concatenate([v[:, 1:, :], v[:, HGRID - 1:, :]], axis=1)
    sy = jnp.concatenate([v[1:, :, :], v[HGRID - 1:, :, :]], axis=0)
    sxy = jnp.concatenate([sx[1:, :, :], sx[HGRID - 1:, :, :]], axis=0)
    o_ref[0, 0] = jnp.concatenate([v, sx, sy, sxy], axis=-1).reshape(NQ, 4 * DH)


def _tc_corner_pack(vplanes):
    return pl.pallas_call(
        _corner_body,
        grid=(NH, NL),
        in_specs=[pl.BlockSpec((1, 1, NQ, DH), lambda h, l: (h, l, 0, 0))],
        out_specs=pl.BlockSpec((1, 1, NQ, 4 * DH), lambda h, l: (h, l, 0, 0)),
        out_shape=jax.ShapeDtypeStruct((NH, NL, NQ, 4 * DH), jnp.float32),
    )(vplanes)


NLANE = NH * NL * NP_


def _samp_body(x_ref, pos_ref, g_ref, b_ref, wso_ref, bso_ref, wa_ref, ba_ref,
               ref_ref, idx_ref, w_ref):
    xq = _ln(x_ref[...] + pos_ref[...], g_ref[...], b_ref[...])
    so = (jnp.dot(xq, wso_ref[...], preferred_element_type=jnp.float32,
                  precision=lax.Precision.HIGHEST)
          + bso_ref[...])
    sx = so[:, :NLANE]
    sy = so[:, NLANE:]
    logits = (jnp.dot(xq, wa_ref[...], preferred_element_type=jnp.float32)
              + ba_ref[...])
    parts = []
    for h in range(NH):
        blk = logits[:, h * 16:(h + 1) * 16]
        m = jnp.max(blk, axis=-1, keepdims=True)
        e = jnp.exp(blk - m)
        parts.append(e / jnp.sum(e, axis=-1, keepdims=True))
    aw = jnp.concatenate(parts, axis=-1)

    lane_l = (lax.broadcasted_iota(jnp.int32, (NL, NLANE), 1) // NP_) % NL
    m4 = (lane_l == lax.broadcasted_iota(jnp.int32, (NL, NLANE), 0)
          ).astype(jnp.float32)
    rx = jnp.dot(ref_ref[..., 0], m4, preferred_element_type=jnp.float32,
                 precision=lax.Precision.HIGHEST)
    ry = jnp.dot(ref_ref[..., 1], m4, preferred_element_type=jnp.float32,
                 precision=lax.Precision.HIGHEST)

    gx = (rx + sx * (1.0 / HGRID)) * HGRID - 0.5
    gy = (ry + sy * (1.0 / HGRID)) * HGRID - 0.5
    x0 = jnp.floor(gx)
    y0 = jnp.floor(gy)
    wx1 = gx - x0
    wx0 = 1.0 - wx1
    wy1 = gy - y0
    wy0 = 1.0 - wy1
    bx = jnp.clip(x0, 0.0, HGRID - 2.0)
    by = jnp.clip(y0, 0.0, HGRID - 2.0)

    lane = lax.broadcasted_iota(jnp.int32, (QBLK, NLANE), 1)
    hl = lane // 16 * NL + (lane // NP_) % NL
    idx_ref[...] = (hl * NQ + by.astype(jnp.int32) * HGRID
                    + bx.astype(jnp.int32))

    rr = lax.broadcasted_iota(jnp.int32, (NLANE, 4 * NLANE), 0)
    cc = lax.broadcasted_iota(jnp.int32, (NLANE, 4 * NLANE), 1)
    acc = jnp.zeros((QBLK, 4 * NLANE), jnp.float32)
    for s, (dy, dx) in enumerate(((0.0, 0.0), (0.0, 1.0), (1.0, 0.0), (1.0, 1.0))):
        sxc = bx + dx
        syc = by + dy
        fx = jnp.where(sxc == x0, wx0, jnp.where(sxc == x0 + 1.0, wx1, 0.0))
        fy = jnp.where(syc == y0, wy0, jnp.where(syc == y0 + 1.0, wy1, 0.0))
        ws = fx * fy * aw
        perm = (cc == (rr // 16) * 64 + s * 16 + rr % 16).astype(jnp.float32)
        acc = acc + jnp.dot(ws, perm, preferred_element_type=jnp.float32)
    w_ref[...] = acc


def _tc_samp(x, pos, g, b, wso, bso, wa, ba, ref):
    return pl.pallas_call(
        _samp_body,
        grid=(NQB,),
        in_specs=[
            pl.BlockSpec((QBLK, C), lambda i: (i, 0)),
            pl.BlockSpec((QBLK, C), lambda i: (i, 0)),
            pl.BlockSpec((1, C), lambda i: (0, 0)),
            pl.BlockSpec((1, C), lambda i: (0, 0)),
            pl.BlockSpec((C, 2 * NLANE), lambda i: (0, 0)),
            pl.BlockSpec((1, 2 * NLANE), lambda i: (0, 0)),
            pl.BlockSpec((C, NLANE), lambda i: (0, 0)),
            pl.BlockSpec((1, NLANE), lambda i: (0, 0)),
            pl.BlockSpec((QBLK, NL, 2), lambda i: (i, 0, 0)),
        ],
        out_specs=[
            pl.BlockSpec((QBLK, NLANE), lambda i: (i, 0)),
            pl.BlockSpec((QBLK, 4 * NLANE), lambda i: (i, 0)),
        ],
        out_shape=[
            jax.ShapeDtypeStruct((NQ, NLANE), jnp.int32),
            jax.ShapeDtypeStruct((NQ, 4 * NLANE), jnp.float32),
        ],
    )(x, pos, g, b, wso, bso, wa, ba, ref)


def _outffn_body(s_ref, wout_ref, bout_ref, x_ref, g_ref, b_ref,
                 w1_ref, b1_ref, w2_ref, b2_ref, o_ref):
    x2 = (jnp.dot(s_ref[...], wout_ref[...], preferred_element_type=jnp.float32)
          + bout_ref[...] + x_ref[...])
    xn = _ln(x2, g_ref[...], b_ref[...])
    h1 = jnp.dot(xn, w1_ref[...], preferred_element_type=jnp.float32) + b1_ref[...]
    ff = jnp.dot(h1, w2_ref[...], preferred_element_type=jnp.float32) + b2_ref[...]
    o_ref[...] = ff + x2


def _tc_outffn(sampled, wout, bout, x, g, b, w1, b1, w2, b2):
    return pl.pallas_call(
        _outffn_body,
        grid=(NQB,),
        in_specs=[
            pl.BlockSpec((QBLK, C), lambda i: (i, 0)),
            pl.BlockSpec((C, C), lambda i: (0, 0)),
            pl.BlockSpec((1, C), lambda i: (0, 0)),
            pl.BlockSpec((QBLK, C), lambda i: (i, 0)),
            pl.BlockSpec((1, C), lambda i: (0, 0)),
            pl.BlockSpec((1, C), lambda i: (0, 0)),
            pl.BlockSpec((C, WF * C), lambda i: (0, 0)),
            pl.BlockSpec((1, WF * C), lambda i: (0, 0)),
            pl.BlockSpec((WF * C, C), lambda i: (0, 0)),
            pl.BlockSpec((1, C), lambda i: (0, 0)),
        ],
        out_specs=pl.BlockSpec((QBLK, C), lambda i: (i, 0)),
        out_shape=jax.ShapeDtypeStruct((NQ, C), jnp.float32),
    )(sampled, wout, bout, x, g, b, w1, b1, w2, b2)


def _sc_body(value_hbm, idx_hbm, w_hbm, out_hbm, idx_v, w_v, rows_v, out_v,
             sem_i, sem_w, sem_r):
    wid = lax.axis_index("s") * 2 + lax.axis_index("c")
    brow = wid * NBLK

    def fire_idx(b, slot):
        pltpu.async_copy(idx_hbm.at[brow + b], idx_v.at[slot], sem_i.at[slot])
        pltpu.async_copy(w_hbm.at[brow + b], w_v.at[slot], sem_w.at[slot])

    def wait_idx(slot):
        pltpu.make_async_copy(idx_hbm.at[0], idx_v.at[slot],
                              sem_i.at[slot]).wait()
        pltpu.make_async_copy(w_hbm.at[0], w_v.at[slot],
                              sem_w.at[slot]).wait()

    def fire_gathers(slot):
        def fj(j, c):
            pltpu.async_copy(value_hbm.at[idx_v.at[slot, pl.ds(j * 16, 16)]],
                             rows_v.at[slot, pl.ds(j * 16, 16)], sem_r.at[slot])
            return c
        lax.fori_loop(0, GB, fj, 0)

    def drain_gathers(slot):
        def dj(j, c):
            pltpu.make_async_copy(
                value_hbm.at[idx_v.at[slot, pl.ds(j * 16, 16)]],
                rows_v.at[slot, pl.ds(j * 16, 16)], sem_r.at[slot]).wait()
            return c
        lax.fori_loop(0, GB, dj, 0)

    def compute_block(b, slot):
        dnums = lax.GatherDimensionNumbers(
            offset_dims=(), collapsed_slice_dims=(0,), start_index_map=(0,))

        def pj(j, c):
            wvecs = [w_v[slot, pl.ds(j * 64 + g * 16, 16)] for g in range(4)]
            acc0 = jnp.zeros((16,), jnp.float32)
            acc1 = jnp.zeros((16,), jnp.float32)
            for i in range(NCONTRIB):
                g, lane = divmod(i, 16)
                ws = lax.gather(
                    wvecs[g], jnp.full((16, 1), lane, jnp.int32), dnums, (1,),
                    mode=lax.GatherScatterMode.PROMISE_IN_BOUNDS)
                r0 = rows_v[slot, j * 16 + i % 16, pl.ds((i // 16) * DH, 16)]
                r1 = rows_v[slot, j * 16 + i % 16, pl.ds((i // 16) * DH + 16, 16)]
                acc0 = acc0 + ws * r0
                acc1 = acc1 + ws * r1
            pit = b * GB + j
            out_v[pit // 4, pl.ds((pit % 4) * DH, 16)] = acc0
            out_v[pit // 4, pl.ds((pit % 4) * DH + 16, 16)] = acc1
            return c
        lax.fori_loop(0, GB, pj, 0)

    fire_idx(0, 0)
    wait_idx(0)
    fire_gathers(0)
    fire_idx(1, 1)

    def body2(t, c):
        b0 = 2 * t
        b1 = 2 * t + 1
        drain_gathers(0)
        wait_idx(1)
        fire_gathers(1)
        compute_block(b0, 0)

        @pl.when(b0 + 2 < NBLK)
        def _():
            fire_idx(b0 + 2, 0)

        drain_gathers(1)

        @pl.when(b1 + 1 < NBLK)
        def _():
            wait_idx(0)
            fire_gathers(0)

        compute_block(b1, 1)

        @pl.when(b1 + 2 < NBLK)
        def _():
            fire_idx(b1 + 2, 1)

        return c

    lax.fori_loop(0, NBLK // 2, body2, 0)
    pltpu.sync_copy(out_v, out_hbm.at[pl.ds(wid * (NPT // 4), NPT // 4)])


def _sc_gather(valtab, idx, w):
    mesh = plsc.VectorSubcoreMesh(core_axis_name="c", subcore_axis_name="s")
    fn = functools.partial(
        pl.kernel,
        out_type=jax.ShapeDtypeStruct((NPAIR // 4, 4 * DH), jnp.float32),
        mesh=mesh,
        scratch_types=[
            pltpu.VMEM((2, GB * 16), jnp.int32),
            pltpu.VMEM((2, GB * NCONTRIB), jnp.float32),
            pltpu.VMEM((2, GB * 16, 4 * DH), jnp.float32),
            pltpu.VMEM((NPT // 4, 4 * DH), jnp.float32),
            pltpu.SemaphoreType.DMA((2,)),
            pltpu.SemaphoreType.DMA((2,)),
            pltpu.SemaphoreType.DMA((2,)),
        ],
    )(_sc_body)
    return fn(valtab, idx, w)


def kernel(query, history_bevs, reference_points, spatial_shapes,
           level_start_index, pos_embedding, params):
    p = params
    q2 = query[0]
    pos2 = pos_embedding[0]
    hist2 = history_bevs[0]

    def r2(v):
        return v.reshape(1, -1)

    qkvh = _tc_qkv(q2, pos2, r2(p['ln1_g']), r2(p['ln1_b']), p['Wqkv'])
    attnh = _tc_attn(qkvh[:NH], qkvh[NH:2 * NH], qkvh[2 * NH:])
    x = _tc_proj_res(attnh, p['Wo'], r2(p['bo']), q2)

    vplanes = _tc_value(hist2, r2(p['ln2_g']), r2(p['ln2_b']), p['Wv'],
                        r2(p['bv']))
    table4 = _tc_corner_pack(vplanes).reshape(NH * NL * NQ, 4 * DH)

    wso_p = jnp.concatenate([p['Wso'][:, 0::2], p['Wso'][:, 1::2]], axis=1)
    bso_p = jnp.concatenate([p['bso'][0::2], p['bso'][1::2]])
    idxq, wq = _tc_samp(x, pos2, r2(p['ln2_g']), r2(p['ln2_b']),
                        wso_p, r2(bso_p), p['Wa'], r2(p['ba']),
                        reference_points[0])

    sampled = _sc_gather(table4, idxq.reshape(NPAIR // GB, GB * 16),
                         wq.reshape(NPAIR // GB, GB * NCONTRIB)).reshape(NQ, C)

    out = _tc_outffn(sampled, p['Wout'], r2(p['bout']), x,
                     r2(p['ln3_g']), r2(p['ln3_b']),
                     p['W1'], r2(p['b1']), p['W2'], r2(p['b2']))
    return out[None]


# speedup vs baseline: 1.5916x; 1.4001x over previous
import functools

import jax
import jax.numpy as jnp
from jax import lax
from jax.experimental import pallas as pl
from jax.experimental.pallas import tpu as pltpu
from jax.experimental.pallas import tpu_sc as plsc

C = 256
NH = 8
DH = C // NH
NL = 4
NP_ = 4
HGRID = 64
NQ = HGRID * HGRID
NV = NL * NQ
WF = 4

QBLK = 512
NQB = NQ // QBLK

NPAIR = NQ * NH
NCONTRIB = NL * NP_ * 4

NTILE = 32
NPT = NPAIR // NTILE
GB = 16
NBLK = NPT // GB


def _ln(x, g, b):
    m = jnp.mean(x, axis=-1, keepdims=True)
    v = jnp.mean((x - m) ** 2, axis=-1, keepdims=True)
    return (x - m) / jnp.sqrt(v + 1e-5) * g + b


def _qkv_body(x_ref, pos_ref, g_ref, b_ref, w_ref, o_ref):
    xn = _ln(x_ref[...] + pos_ref[...], g_ref[...], b_ref[...])
    res = jnp.dot(xn, w_ref[...], preferred_element_type=jnp.float32)
    for k in range(3 * NH):
        blk = res[:, k * DH:(k + 1) * DH]
        if k < NH:
            blk = blk * (DH ** -0.5)
        o_ref[k] = blk.astype(jnp.bfloat16)


def _tc_qkv(x, pos, g, b, w):
    return pl.pallas_call(
        _qkv_body,
        grid=(NQB,),
        in_specs=[
            pl.BlockSpec((QBLK, C), lambda i: (i, 0)),
            pl.BlockSpec((QBLK, C), lambda i: (i, 0)),
            pl.BlockSpec((1, C), lambda i: (0, 0)),
            pl.BlockSpec((1, C), lambda i: (0, 0)),
            pl.BlockSpec((C, 3 * C), lambda i: (0, 0)),
        ],
        out_specs=pl.BlockSpec((3 * NH, QBLK, DH), lambda i: (0, i, 0)),
        out_shape=jax.ShapeDtypeStruct((3 * NH, NQ, DH), jnp.bfloat16),
    )(x, pos, g, b, w)


def _attn_body(q_ref, k_ref, v_ref, o_ref):
    q = q_ref[0]
    k = k_ref[0]
    s = lax.dot_general(q, k, (((1,), (1,)), ((), ())),
                        preferred_element_type=jnp.float32)
    e = jnp.exp(s)
    z = jnp.sum(e, axis=-1, keepdims=True)
    o = jnp.dot(e.astype(jnp.bfloat16), v_ref[0],
                preferred_element_type=jnp.float32)
    o_ref[0] = o / z


def _tc_attn(qh, kh, vh):
    return pl.pallas_call(
        _attn_body,
        grid=(NH, NQB),
        in_specs=[
            pl.BlockSpec((1, QBLK, DH), lambda h, i: (h, i, 0)),
            pl.BlockSpec((1, NQ, DH), lambda h, i: (h, 0, 0)),
            pl.BlockSpec((1, NQ, DH), lambda h, i: (h, 0, 0)),
        ],
        out_specs=pl.BlockSpec((1, QBLK, DH), lambda h, i: (h, i, 0)),
        out_shape=jax.ShapeDtypeStruct((NH, NQ, DH), jnp.float32),
    )(qh, kh, vh)


def _proj_res_body(a_ref, w_ref, b_ref, r_ref, o_ref):
    a = jnp.concatenate([a_ref[h] for h in range(NH)], axis=-1)
    o_ref[...] = (jnp.dot(a, w_ref[...], preferred_element_type=jnp.float32)
                  + b_ref[...] + r_ref[...])


def _tc_proj_res(attnh, w, b, res):
    return pl.pallas_call(
        _proj_res_body,
        grid=(NQB,),
        in_specs=[
            pl.BlockSpec((NH, QBLK, DH), lambda i: (0, i, 0)),
            pl.BlockSpec((C, C), lambda i: (0, 0)),
            pl.BlockSpec((1, C), lambda i: (0, 0)),
            pl.BlockSpec((QBLK, C), lambda i: (i, 0)),
        ],
        out_specs=pl.BlockSpec((QBLK, C), lambda i: (i, 0)),
        out_shape=jax.ShapeDtypeStruct((NQ, C), jnp.float32),
    )(attnh, w, b, res)


def _value_body(h_ref, g_ref, b_ref, w_ref, bv_ref, o_ref):
    xn = _ln(h_ref[...], g_ref[...], b_ref[...])
    res = jnp.dot(xn, w_ref[...], preferred_element_type=jnp.float32) + bv_ref[...]
    for h in range(NH):
        o_ref[h, 0] = res[:, h * DH:(h + 1) * DH]


def _tc_value(hist, g, b, w, bv):
    blk = 1024
    return pl.pallas_call(
        _value_body,
        grid=(NV // blk,),
        in_specs=[
            pl.BlockSpec((blk, C), lambda i: (i, 0)),
            pl.BlockSpec((1, C), lambda i: (0, 0)),
            pl.BlockSpec((1, C), lambda i: (0, 0)),
            pl.BlockSpec((C, C), lambda i: (0, 0)),
            pl.BlockSpec((1, C), lambda i: (0, 0)),
        ],
        out_specs=pl.BlockSpec((NH, 1, blk, DH), lambda i: (0, i // 4, i % 4, 0)),
        out_shape=jax.ShapeDtypeStruct((NH, NL, NQ, DH), jnp.float32),
    )(hist, g, b, w, bv)


def _corner_body(v_ref, o_ref):
    v = v_ref[0, 0].reshape(HGRID, HGRID, DH)
    sx = jnp.concatenate([v[:, 1:, :], v[:, HGRID - 1:, :]], axis=1)
    sy = jnp.concatenate([v[1:, :, :], v[HGRID - 1:, :, :]], axis=0)
    sxy = jnp.concatenate([sx[1:, :, :], sx[HGRID - 1:, :, :]], axis=0)
    o_ref[0, 0] = jnp.concatenate([v, sx, sy, sxy], axis=-1).reshape(NQ, 4 * DH)


def _tc_corner_pack(vplanes):
    return pl.pallas_call(
        _corner_body,
        grid=(NH, NL),
        in_specs=[pl.BlockSpec((1, 1, NQ, DH), lambda h, l: (h, l, 0, 0))],
        out_specs=pl.BlockSpec((1, 1, NQ, 4 * DH), lambda h, l: (h, l, 0, 0)),
        out_shape=jax.ShapeDtypeStruct((NH, NL, NQ, 4 * DH), jnp.float32),
    )(vplanes)


NLANE = NH * NL * NP_


def _samp_body(x_ref, pos_ref, g_ref, b_ref, wso_ref, bso_ref, wa_ref, ba_ref,
               ref_ref, idx_ref, w_ref):
    xq = _ln(x_ref[...] + pos_ref[...], g_ref[...], b_ref[...])
    so = (jnp.dot(xq, wso_ref[...], preferred_element_type=jnp.float32,
                  precision=lax.Precision.HIGHEST)
          + bso_ref[...])
    sx = so[:, :NLANE]
    sy = so[:, NLANE:]
    logits = (jnp.dot(xq, wa_ref[...], preferred_element_type=jnp.float32)
              + ba_ref[...])
    parts = []
    for h in range(NH):
        blk = logits[:, h * 16:(h + 1) * 16]
        m = jnp.max(blk, axis=-1, keepdims=True)
        e = jnp.exp(blk - m)
        parts.append(e / jnp.sum(e, axis=-1, keepdims=True))
    aw = jnp.concatenate(parts, axis=-1)

    lane_l = (lax.broadcasted_iota(jnp.int32, (NL, NLANE), 1) // NP_) % NL
    m4 = (lane_l == lax.broadcasted_iota(jnp.int32, (NL, NLANE), 0)
          ).astype(jnp.float32)
    rx = jnp.dot(ref_ref[..., 0], m4, preferred_element_type=jnp.float32,
                 precision=lax.Precision.HIGHEST)
    ry = jnp.dot(ref_ref[..., 1], m4, preferred_element_type=jnp.float32,
                 precision=lax.Precision.HIGHEST)

    gx = (rx + sx * (1.0 / HGRID)) * HGRID - 0.5
    gy = (ry + sy * (1.0 / HGRID)) * HGRID - 0.5
    x0 = jnp.floor(gx)
    y0 = jnp.floor(gy)
    wx1 = gx - x0
    wx0 = 1.0 - wx1
    wy1 = gy - y0
    wy0 = 1.0 - wy1
    bx = jnp.clip(x0, 0.0, HGRID - 2.0)
    by = jnp.clip(y0, 0.0, HGRID - 2.0)

    lane = lax.broadcasted_iota(jnp.int32, (QBLK, NLANE), 1)
    hl = lane // 16 * NL + (lane // NP_) % NL
    idx_ref[...] = (hl * NQ + by.astype(jnp.int32) * HGRID
                    + bx.astype(jnp.int32))

    rr = lax.broadcasted_iota(jnp.int32, (NLANE, 4 * NLANE), 0)
    cc = lax.broadcasted_iota(jnp.int32, (NLANE, 4 * NLANE), 1)
    acc = jnp.zeros((QBLK, 4 * NLANE), jnp.float32)
    for s, (dy, dx) in enumerate(((0.0, 0.0), (0.0, 1.0), (1.0, 0.0), (1.0, 1.0))):
        sxc = bx + dx
        syc = by + dy
        fx = jnp.where(sxc == x0, wx0, jnp.where(sxc == x0 + 1.0, wx1, 0.0))
        fy = jnp.where(syc == y0, wy0, jnp.where(syc == y0 + 1.0, wy1, 0.0))
        ws = fx * fy * aw
        perm = (cc == (rr // 16) * 64 + s * 16 + rr % 16).astype(jnp.float32)
        acc = acc + jnp.dot(ws, perm, preferred_element_type=jnp.float32)
    w_ref[...] = acc


def _tc_samp(x, pos, g, b, wso, bso, wa, ba, ref):
    return pl.pallas_call(
        _samp_body,
        grid=(NQB,),
        in_specs=[
            pl.BlockSpec((QBLK, C), lambda i: (i, 0)),
            pl.BlockSpec((QBLK, C), lambda i: (i, 0)),
            pl.BlockSpec((1, C), lambda i: (0, 0)),
            pl.BlockSpec((1, C), lambda i: (0, 0)),
            pl.BlockSpec((C, 2 * NLANE), lambda i: (0, 0)),
            pl.BlockSpec((1, 2 * NLANE), lambda i: (0, 0)),
            pl.BlockSpec((C, NLANE), lambda i: (0, 0)),
            pl.BlockSpec((1, NLANE), lambda i: (0, 0)),
            pl.BlockSpec((QBLK, NL, 2), lambda i: (i, 0, 0)),
        ],
        out_specs=[
            pl.BlockSpec((QBLK, NLANE), lambda i: (i, 0)),
            pl.BlockSpec((QBLK, 4 * NLANE), lambda i: (i, 0)),
        ],
        out_shape=[
            jax.ShapeDtypeStruct((NQ, NLANE), jnp.int32),
            jax.ShapeDtypeStruct((NQ, 4 * NLANE), jnp.float32),
        ],
    )(x, pos, g, b, wso, bso, wa, ba, ref)


def _outffn_body(s_ref, wout_ref, bout_ref, x_ref, g_ref, b_ref,
                 w1_ref, b1_ref, w2_ref, b2_ref, o_ref):
    x2 = (jnp.dot(s_ref[...], wout_ref[...], preferred_element_type=jnp.float32)
          + bout_ref[...] + x_ref[...])
    xn = _ln(x2, g_ref[...], b_ref[...])
    h1 = jnp.dot(xn, w1_ref[...], preferred_element_type=jnp.float32) + b1_ref[...]
    ff = jnp.dot(h1, w2_ref[...], preferred_element_type=jnp.float32) + b2_ref[...]
    o_ref[...] = ff + x2


def _tc_outffn(sampled, wout, bout, x, g, b, w1, b1, w2, b2):
    return pl.pallas_call(
        _outffn_body,
        grid=(NQB,),
        in_specs=[
            pl.BlockSpec((QBLK, C), lambda i: (i, 0)),
            pl.BlockSpec((C, C), lambda i: (0, 0)),
            pl.BlockSpec((1, C), lambda i: (0, 0)),
            pl.BlockSpec((QBLK, C), lambda i: (i, 0)),
            pl.BlockSpec((1, C), lambda i: (0, 0)),
            pl.BlockSpec((1, C), lambda i: (0, 0)),
            pl.BlockSpec((C, WF * C), lambda i: (0, 0)),
            pl.BlockSpec((1, WF * C), lambda i: (0, 0)),
            pl.BlockSpec((WF * C, C), lambda i: (0, 0)),
            pl.BlockSpec((1, C), lambda i: (0, 0)),
        ],
        out_specs=pl.BlockSpec((QBLK, C), lambda i: (i, 0)),
        out_shape=jax.ShapeDtypeStruct((NQ, C), jnp.float32),
    )(sampled, wout, bout, x, g, b, w1, b1, w2, b2)


def _sc_body(value_hbm, idx_hbm, w_hbm, out_hbm, idx_v, w_v, rows_v, out_v,
             sem_i, sem_w, sem_r):
    wid = lax.axis_index("s") * 2 + lax.axis_index("c")
    brow = wid * NBLK

    def fire_idx(b, slot):
        pltpu.async_copy(idx_hbm.at[brow + b], idx_v.at[slot], sem_i.at[slot])
        pltpu.async_copy(w_hbm.at[brow + b], w_v.at[slot], sem_w.at[slot])

    def wait_idx(slot):
        pltpu.make_async_copy(idx_hbm.at[0], idx_v.at[slot],
                              sem_i.at[slot]).wait()
        pltpu.make_async_copy(w_hbm.at[0], w_v.at[slot],
                              sem_w.at[slot]).wait()

    def fire_gathers(slot):
        def fj(j, c):
            pltpu.async_copy(value_hbm.at[idx_v.at[slot, pl.ds(j * 16, 16)]],
                             rows_v.at[slot, pl.ds(j * 16, 16)], sem_r.at[slot])
            return c
        lax.fori_loop(0, GB, fj, 0)

    def drain_gathers(slot):
        def dj(j, c):
            pltpu.make_async_copy(
                value_hbm.at[idx_v.at[slot, pl.ds(j * 16, 16)]],
                rows_v.at[slot, pl.ds(j * 16, 16)], sem_r.at[slot]).wait()
            return c
        lax.fori_loop(0, GB, dj, 0)

    def compute_block(b, slot):
        dnums = lax.GatherDimensionNumbers(
            offset_dims=(), collapsed_slice_dims=(0,), start_index_map=(0,))

        def pj(j, c):
            wvecs = [w_v[slot, pl.ds(j * 64 + g * 16, 16)] for g in range(4)]
            acc0 = jnp.zeros((16,), jnp.float32)
            acc1 = jnp.zeros((16,), jnp.float32)
            for i in range(NCONTRIB):
                g, lane = divmod(i, 16)
                ws = lax.gather(
                    wvecs[g], jnp.full((16, 1), lane, jnp.int32), dnums, (1,),
                    mode=lax.GatherScatterMode.PROMISE_IN_BOUNDS)
                r0 = rows_v[slot, j * 16 + i % 16, pl.ds((i // 16) * DH, 16)]
                r1 = rows_v[slot, j * 16 + i % 16, pl.ds((i // 16) * DH + 16, 16)]
                acc0 = acc0 + ws * r0
                acc1 = acc1 + ws * r1
            pit = b * GB + j
            out_v[pit // 4, pl.ds((pit % 4) * DH, 16)] = acc0
            out_v[pit // 4, pl.ds((pit % 4) * DH + 16, 16)] = acc1
            return c
        lax.fori_loop(0, GB, pj, 0)

    fire_idx(0, 0)
    wait_idx(0)
    fire_gathers(0)
    fire_idx(1, 1)

    def body2(t, c):
        b0 = 2 * t
        b1 = 2 * t + 1
        drain_gathers(0)
        wait_idx(1)
        fire_gathers(1)
        compute_block(b0, 0)

        @pl.when(b0 + 2 < NBLK)
        def _():
            fire_idx(b0 + 2, 0)

        drain_gathers(1)

        @pl.when(b1 + 1 < NBLK)
        def _():
            wait_idx(0)
            fire_gathers(0)

        compute_block(b1, 1)

        @pl.when(b1 + 2 < NBLK)
        def _():
            fire_idx(b1 + 2, 1)

        return c

    lax.fori_loop(0, NBLK // 2, body2, 0)
    pltpu.sync_copy(out_v, out_hbm.at[pl.ds(wid * (NPT // 4), NPT // 4)])


def _sc_gather(valtab, idx, w):
    mesh = plsc.VectorSubcoreMesh(core_axis_name="c", subcore_axis_name="s")
    fn = functools.partial(
        pl.kernel,
        out_type=jax.ShapeDtypeStruct((NPAIR // 4, 4 * DH), jnp.float32),
        mesh=mesh,
        scratch_types=[
            pltpu.VMEM((2, GB * 16), jnp.int32),
            pltpu.VMEM((2, GB * NCONTRIB), jnp.float32),
            pltpu.VMEM((2, GB * 16, 4 * DH), jnp.float32),
            pltpu.VMEM((NPT // 4, 4 * DH), jnp.float32),
            pltpu.SemaphoreType.DMA((2,)),
            pltpu.SemaphoreType.DMA((2,)),
            pltpu.SemaphoreType.DMA((2,)),
        ],
    )(_sc_body)
    return fn(valtab, idx, w)


def kernel(query, history_bevs, reference_points, spatial_shapes,
           level_start_index, pos_embedding, params):
    p = params
    q2 = query[0]
    pos2 = pos_embedding[0]
    hist2 = history_bevs[0]

    def r2(v):
        return v.reshape(1, -1)

    qkvh = _tc_qkv(q2, pos2, r2(p['ln1_g']), r2(p['ln1_b']), p['Wqkv'])
    attnh = _tc_attn(qkvh[:NH], qkvh[NH:2 * NH], qkvh[2 * NH:])
    x = _tc_proj_res(attnh, p['Wo'], r2(p['bo']), q2)

    vplanes = _tc_value(hist2, r2(p['ln2_g']), r2(p['ln2_b']), p['Wv'],
                        r2(p['bv']))
    table4 = _tc_corner_pack(vplanes).reshape(NH * NL * NQ, 4 * DH)

    wso_p = jnp.concatenate([p['Wso'][:, 0::2], p['Wso'][:, 1::2]], axis=1)
    bso_p = jnp.concatenate([p['bso'][0::2], p['bso'][1::2]])
    idxq, wq = _tc_samp(x, pos2, r2(p['ln2_g']), r2(p['ln2_b']),
                        wso_p, r2(bso_p), p['Wa'], r2(p['ba']),
                        reference_points[0])

    sampled = _sc_gather(table4, idxq.reshape(NPAIR // GB, GB * 16),
                         wq.reshape(NPAIR // GB, GB * NCONTRIB)).reshape(NQ, C)

    out = _tc_outffn(sampled, p['Wout'], r2(p['bout']), x,
                     r2(p['ln3_g']), r2(p['ln3_b']),
                     p['W1'], r2(p['b1']), p['W2'], r2(p['b2']))
    return out[None]
